# Initial kernel scaffold; baseline (speedup 1.0000x reference)
import jax
import jax.numpy as jnp
from jax.experimental import pallas as pl


def kernel(x, shared_table, task_tables, W1, b1, W2, b2, W3, b3, tw1, tb1, tw2, tb2):
    raise NotImplementedError("write your pallas kernel here")


# speedup vs baseline: 5.5647x; 5.5647x over previous
import functools

import jax
import jax.numpy as jnp
from jax import lax
from jax.experimental import pallas as pl
from jax.experimental.pallas import tpu as pltpu
from jax.experimental.pallas import tpu_sc as plsc

B, F, V, D = 16384, 26, 100000, 64
T = 2
FD = F * D
IN_DIM = 2 * FD
BF = B * F

NC, NS = 2, 16
NW = NC * NS
ROWS_PER_W = BF // NW
CH = 128
NCHUNK = ROWS_PER_W // CH

TB = 1024


def _sc_gather(idx, idxv, shared_table, task2):
    mesh = plsc.VectorSubcoreMesh(core_axis_name="c", subcore_axis_name="s")
    out_t = [jax.ShapeDtypeStruct((BF, D), jnp.float32)] * 3

    @functools.partial(
        pl.kernel,
        mesh=mesh,
        out_type=out_t,
        scratch_types=[
            pltpu.VMEM((CH,), jnp.int32),
            pltpu.VMEM((CH,), jnp.int32),
            pltpu.VMEM((CH, D), jnp.float32),
            pltpu.VMEM((CH, D), jnp.float32),
            pltpu.VMEM((CH, D), jnp.float32),
            pltpu.SemaphoreType.DMA,
        ],
        compiler_params=pltpu.CompilerParams(use_tc_tiling_on_sc=False),
    )
    def k(idx_hbm, idxv_hbm, s_hbm, t2_hbm, os_hbm, o0_hbm, o1_hbm,
          i0_v, i1_v, rs, r0, r1, sem):
        wid = lax.axis_index("s") * NC + lax.axis_index("c")
        base = wid * ROWS_PER_W

        @pl.loop(0, NCHUNK)
        def _(c):
            off = base + c * CH
            pltpu.sync_copy(idx_hbm.at[pl.ds(off, CH)], i0_v)
            pltpu.sync_copy(idxv_hbm.at[pl.ds(off, CH)], i1_v)
            cs = pltpu.async_copy(s_hbm.at[i0_v], rs, sem)
            c0 = pltpu.async_copy(t2_hbm.at[i0_v], r0, sem)
            c1 = pltpu.async_copy(t2_hbm.at[i1_v], r1, sem)
            cs.wait()
            c0.wait()
            c1.wait()
            pltpu.sync_copy(rs, os_hbm.at[pl.ds(off, CH)])
            pltpu.sync_copy(r0, o0_hbm.at[pl.ds(off, CH)])
            pltpu.sync_copy(r1, o1_hbm.at[pl.ds(off, CH)])

    return k(idx, idxv, shared_table, task2)


def _dense_body(gs_ref, g0_ref, g1_ref, w1_ref, b1_ref, w2_ref, b2_ref,
                w3_ref, b3_ref, tw1_ref, tb1_ref, tw2_ref, tb2_ref, out_ref):
    w1s = w1_ref[0:FD, :]
    w1t = w1_ref[FD:IN_DIM, :]
    a_s = jnp.dot(gs_ref[...], w1s, preferred_element_type=jnp.float32) + b1_ref[...]
    logits = []
    for i in range(T):
        gi = g0_ref[...] if i == 0 else g1_ref[...]
        h = jnp.maximum(a_s + jnp.dot(gi, w1t, preferred_element_type=jnp.float32), 0.0)
        h = jnp.maximum(jnp.dot(h, w2_ref[...], preferred_element_type=jnp.float32) + b2_ref[...], 0.0)
        h = jnp.maximum(jnp.dot(h, w3_ref[...], preferred_element_type=jnp.float32) + b3_ref[...], 0.0)
        t = jnp.maximum(jnp.dot(h, tw1_ref[i], preferred_element_type=jnp.float32) + tb1_ref[i:i + 1, :], 0.0)
        logit = jnp.sum(t * tw2_ref[i:i + 1, :], axis=1, keepdims=True) + tb2_ref[i:i + 1, :]
        logits.append(logit)
    out_ref[...] = jax.nn.sigmoid(jnp.concatenate(logits, axis=1))


def _tc_dense(gs, g0, g1, W1, b1, W2, b2, W3, b3, tw1, tb1, tw2r, tb2):
    full = lambda shape: pl.BlockSpec(shape, lambda i: (0,) * len(shape))
    return pl.pallas_call(
        _dense_body,
        grid=(B // TB,),
        in_specs=[
            pl.BlockSpec((TB, FD), lambda i: (i, 0)),
            pl.BlockSpec((TB, FD), lambda i: (i, 0)),
            pl.BlockSpec((TB, FD), lambda i: (i, 0)),
            full((IN_DIM, 256)),
            full((1, 256)),
            full((256, 128)),
            full((1, 128)),
            full((128, 64)),
            full((1, 64)),
            full((T, 64, 64)),
            full((T, 64)),
            full((T, 64)),
            full((T, 1)),
        ],
        out_specs=pl.BlockSpec((TB, T), lambda i: (i, 0)),
        out_shape=jax.ShapeDtypeStruct((B, T), jnp.float32),
    )(gs, g0, g1, W1, b1, W2, b2, W3, b3, tw1, tb1, tw2r, tb2)


def kernel(x, shared_table, task_tables, W1, b1, W2, b2, W3, b3, tw1, tb1, tw2, tb2):
    idx = x.reshape(BF)
    idxv = idx + V
    task2 = task_tables.reshape(2 * V, D)
    gs, g0, g1 = _sc_gather(idx, idxv, shared_table, task2)
    return _tc_dense(
        gs.reshape(B, FD), g0.reshape(B, FD), g1.reshape(B, FD),
        W1, b1.reshape(1, 256), W2, b2.reshape(1, 128), W3, b3.reshape(1, 64),
        tw1, tb1, tw2.reshape(T, 64), tb2,
    )


# speedup vs baseline: 5.7950x; 1.0414x over previous
import functools

import jax
import jax.numpy as jnp
from jax import lax
from jax.experimental import pallas as pl
from jax.experimental.pallas import tpu as pltpu
from jax.experimental.pallas import tpu_sc as plsc

B, F, V, D = 16384, 26, 100000, 64
T = 2
FD = F * D
IN_DIM = 2 * FD
BF = B * F

NC, NS = 2, 16
NW = NC * NS
ROWS_PER_W = BF // NW
CH = 128
NCHUNK = ROWS_PER_W // CH

TB = 1024


def _sc_gather(idx, shared_table, task_tables):
    mesh = plsc.VectorSubcoreMesh(core_axis_name="c", subcore_axis_name="s")
    out_t = [jax.ShapeDtypeStruct((BF, D), jnp.float32)] * 3

    @functools.partial(
        pl.kernel,
        mesh=mesh,
        out_type=out_t,
        scratch_types=[
            pltpu.VMEM((CH,), jnp.int32),
            pltpu.VMEM((CH, D), jnp.float32),
            pltpu.VMEM((CH, D), jnp.float32),
            pltpu.VMEM((CH, D), jnp.float32),
            pltpu.SemaphoreType.DMA,
        ],
        compiler_params=pltpu.CompilerParams(use_tc_tiling_on_sc=False),
    )
    def k(idx_hbm, s_hbm, t_hbm, os_hbm, o0_hbm, o1_hbm,
          i0_v, rs, r0, r1, sem):
        wid = lax.axis_index("s") * NC + lax.axis_index("c")
        base = wid * ROWS_PER_W
        t0_hbm = t_hbm.at[0]
        t1_hbm = t_hbm.at[1]

        @pl.loop(0, NCHUNK)
        def _(c):
            off = base + c * CH
            pltpu.sync_copy(idx_hbm.at[pl.ds(off, CH)], i0_v)
            cs = pltpu.async_copy(s_hbm.at[i0_v], rs, sem)
            c0 = pltpu.async_copy(t0_hbm.at[i0_v], r0, sem)
            c1 = pltpu.async_copy(t1_hbm.at[i0_v], r1, sem)
            cs.wait()
            c0.wait()
            c1.wait()
            pltpu.sync_copy(rs, os_hbm.at[pl.ds(off, CH)])
            pltpu.sync_copy(r0, o0_hbm.at[pl.ds(off, CH)])
            pltpu.sync_copy(r1, o1_hbm.at[pl.ds(off, CH)])

    return k(idx, shared_table, task_tables)


def _dense_body(gs_ref, g0_ref, g1_ref, w1_ref, b1_ref, w2_ref, b2_ref,
                w3_ref, b3_ref, tw1_ref, tb1_ref, tw2_ref, tb2_ref, out_ref):
    w1s = w1_ref[0:FD, :]
    w1t = w1_ref[FD:IN_DIM, :]
    a_s = jnp.dot(gs_ref[...], w1s, preferred_element_type=jnp.float32) + b1_ref[...]
    logits = []
    for i in range(T):
        gi = g0_ref[...] if i == 0 else g1_ref[...]
        h = jnp.maximum(a_s + jnp.dot(gi, w1t, preferred_element_type=jnp.float32), 0.0)
        h = jnp.maximum(jnp.dot(h, w2_ref[...], preferred_element_type=jnp.float32) + b2_ref[...], 0.0)
        h = jnp.maximum(jnp.dot(h, w3_ref[...], preferred_element_type=jnp.float32) + b3_ref[...], 0.0)
        t = jnp.maximum(jnp.dot(h, tw1_ref[i], preferred_element_type=jnp.float32) + tb1_ref[i:i + 1, :], 0.0)
        logit = jnp.sum(t * tw2_ref[i:i + 1, :], axis=1, keepdims=True) + tb2_ref[i:i + 1, :]
        logits.append(logit)
    out_ref[...] = jax.nn.sigmoid(jnp.concatenate(logits, axis=1))


def _tc_dense(gs, g0, g1, W1, b1, W2, b2, W3, b3, tw1, tb1, tw2r, tb2):
    full = lambda shape: pl.BlockSpec(shape, lambda i: (0,) * len(shape))
    return pl.pallas_call(
        _dense_body,
        grid=(B // TB,),
        in_specs=[
            pl.BlockSpec((TB, FD), lambda i: (i, 0)),
            pl.BlockSpec((TB, FD), lambda i: (i, 0)),
            pl.BlockSpec((TB, FD), lambda i: (i, 0)),
            full((IN_DIM, 256)),
            full((1, 256)),
            full((256, 128)),
            full((1, 128)),
            full((128, 64)),
            full((1, 64)),
            full((T, 64, 64)),
            full((T, 64)),
            full((T, 64)),
            full((T, 1)),
        ],
        out_specs=pl.BlockSpec((TB, T), lambda i: (i, 0)),
        out_shape=jax.ShapeDtypeStruct((B, T), jnp.float32),
    )(gs, g0, g1, W1, b1, W2, b2, W3, b3, tw1, tb1, tw2r, tb2)


def kernel(x, shared_table, task_tables, W1, b1, W2, b2, W3, b3, tw1, tb1, tw2, tb2):
    idx = x.reshape(BF)
    gs, g0, g1 = _sc_gather(idx, shared_table, task_tables)
    return _tc_dense(
        gs.reshape(B, FD), g0.reshape(B, FD), g1.reshape(B, FD),
        W1, b1.reshape(1, 256), W2, b2.reshape(1, 128), W3, b3.reshape(1, 64),
        tw1, tb1, tw2.reshape(T, 64), tb2,
    )
